# baseline (speedup 1.0000x reference)
import functools

import jax
import jax.numpy as jnp
from jax import lax
from jax.experimental import pallas as pl
from jax.experimental.pallas import tpu as pltpu
from jax.experimental.pallas import tpu_sc as plsc

_N = 10000
_E = 320000
_H = 128
_L = 3
_G = 256

_NPAD = 10240
_NC = 2
_NS = 16
_NW = _NC * _NS
_K = 80
_C = 128
_RB = 4
_GD = 3
_IB = 8
_EPAD = _NW * _C * _K

_PK = 80
_PC = 4
_PROWS = 384
_PSTRIPE = _PROWS // _NS

_STRIPE = _NPAD // _NS


def _sc_mesh():
    return plsc.VectorSubcoreMesh(
        core_axis_name="c", subcore_axis_name="s", num_cores=_NC, num_subcores=_NS
    )


def _edge_agg_body(hb32_hbm, eidx_hbm, zeros_hbm, out_hbm, idx_v, rbf_v, rf32_v,
                   semi, semg, sems, agg_sp):
    c = lax.axis_index("c")
    s = lax.axis_index("s")
    w = c * _NS + s
    row0 = s * _STRIPE
    nz = _STRIPE // _K
    for t in range(nz):
        pltpu.async_copy(zeros_hbm, agg_sp.at[pl.ds(row0 + t * _K, _K)], sems.at[t % 2])

    def idx_load(j, slot):
        pltpu.async_copy(eidx_hbm.at[w, j], idx_v.at[slot], semi.at[slot])

    def idx_wait(j, slot):
        pltpu.make_async_copy(eidx_hbm.at[w, j], idx_v.at[slot], semi.at[slot]).wait()

    def gather(j, u8, r):
        pltpu.async_copy(hb32_hbm.at[idx_v.at[u8, 0]], rbf_v.at[r], semg.at[r])

    def gather_wait(u8, r):
        pltpu.make_async_copy(hb32_hbm.at[idx_v.at[u8, 0]], rbf_v.at[r], semg.at[r]).wait()

    def scatter(u8, p):
        pltpu.async_copy(rf32_v.at[p], agg_sp.at[idx_v.at[u8, 1]], sems.at[p], add=True)

    def scatter_wait(p):
        pltpu.make_async_copy(rf32_v.at[p], agg_sp.at[idx_v.at[0, 1]], sems.at[p]).wait()

    def convert(r, p):
        rb = rbf_v.at[r]
        rf = rf32_v.at[p]
        himask = jnp.int32(-65536)

        def crow(rr, carry):
            for k in range(4):
                row = rr * 4 + k
                for g in range(_H // 32):
                    xu = rb[row, pl.ds(16 * g, 16)]
                    rf[row, pl.ds(32 * g, 16)] = jax.lax.bitcast_convert_type(xu << 16, jnp.float32)
                    rf[row, pl.ds(32 * g + 16, 16)] = jax.lax.bitcast_convert_type(xu & himask, jnp.float32)
            return carry

        lax.fori_loop(0, _K // 4, crow, 0)

    def iteration(j, u, skip_b=False, skip_ef=False, do_idx=True):
        idx_wait(j, u)
        if not skip_b:
            scatter_wait(u % 2)
        gather(j, u, u % _RB)
        if do_idx:
            idx_load(j + _RB, (u + _RB) % _IB)
        if not skip_ef:
            um = (u - _GD) % _IB
            gather_wait(um, um % _RB)
            convert(um % _RB, um % 2)
            scatter(um, um % 2)

    for t in range(_RB):
        idx_load(t, t)
    for t in range(_GD):
        iteration(t, t, skip_b=True, skip_ef=True)
    for t in range(nz):
        pltpu.make_async_copy(zeros_hbm, agg_sp.at[pl.ds(row0 + t * _K, _K)], sems.at[t % 2]).wait()
    plsc.subcore_barrier()
    for t in range(_GD, _IB):
        iteration(t, t, skip_b=(t < _RB))

    def body(bb, carry):
        j0 = _IB + _IB * bb
        for u in range(_IB):
            iteration(j0 + u, u)
        return carry

    lax.fori_loop(0, (_C - 2 * _IB) // _IB, body, 0)
    for t in range(_IB):
        j = _C - _IB + t
        iteration(j, t, do_idx=(t < _IB - _RB))
    for t in range(_GD):
        j = _C - _GD + t
        u = j % _IB
        gather_wait(u, u % _RB)
        convert(u % _RB, u % 2)
        scatter(u, u % 2)
    for p in (0, 1, 0, 1):
        scatter_wait(p)
    plsc.subcore_barrier()
    pltpu.sync_copy(agg_sp.at[pl.ds(row0, _STRIPE)], out_hbm.at[c, pl.ds(row0, _STRIPE)])


def _edge_agg(hb32, eidx, zeros_k):
    return pl.kernel(
        _edge_agg_body,
        out_type=jax.ShapeDtypeStruct((_NC, _NPAD, _H), jnp.float32),
        mesh=_sc_mesh(),
        compiler_params=pltpu.CompilerParams(use_tc_tiling_on_sc=False),
        scratch_types=[
            pltpu.VMEM((_IB, 2, _K), jnp.int32),
            pltpu.VMEM((_RB, _K, _H // 2), jnp.int32),
            pltpu.VMEM((2, _K, _H), jnp.float32),
            pltpu.SemaphoreType.DMA((_IB,)),
            pltpu.SemaphoreType.DMA((_RB,)),
            pltpu.SemaphoreType.DMA((2,)),
            pltpu.VMEM_SHARED((_NPAD, _H), jnp.float32),
        ],
    )(hb32, eidx, zeros_k)


_BLK = 1024


def _dense_in_body(x_ref, w_ref, b_ref, o_ref, obf_ref):
    y = (
        jnp.dot(x_ref[...], w_ref[...], preferred_element_type=jnp.float32)
        + b_ref[...]
    )
    o_ref[...] = y
    obf_ref[...] = y.astype(jnp.bfloat16)


def _dense_in(x, w, b):
    return pl.pallas_call(
        _dense_in_body,
        grid=(_NPAD // _BLK,),
        in_specs=[
            pl.BlockSpec((_BLK, _H), lambda i: (i, 0)),
            pl.BlockSpec((_H, _H), lambda i: (0, 0)),
            pl.BlockSpec((1, _H), lambda i: (0, 0)),
        ],
        out_specs=[
            pl.BlockSpec((_BLK, _H), lambda i: (i, 0)),
            pl.BlockSpec((_BLK, _H), lambda i: (i, 0)),
        ],
        out_shape=[
            jax.ShapeDtypeStruct((_NPAD, _H), jnp.float32),
            jax.ShapeDtypeStruct((_NPAD, _H), jnp.bfloat16),
        ],
    )(x, w, b.reshape(1, _H))


def _conv_body(h_ref, a_ref, wr_ref, wn_ref, b_ref, g_ref, bb_ref, o_ref, obf_ref):
    h = h_ref[...]
    a = a_ref[0] + a_ref[1]
    y = (
        jnp.dot(h, wr_ref[...], preferred_element_type=jnp.float32)
        + jnp.dot(a, wn_ref[...], preferred_element_type=jnp.float32)
        + b_ref[...]
    )
    mu = jnp.mean(y, axis=-1, keepdims=True)
    d = y - mu
    var = jnp.mean(d * d, axis=-1, keepdims=True)
    y = d * lax.rsqrt(var + 1e-5) * g_ref[...] + bb_ref[...]
    y = jnp.maximum(y, 0.0)
    o_ref[...] = y
    obf_ref[...] = y.astype(jnp.bfloat16)


def _conv(h, agg2, wr, wn, b, g, bb):
    return pl.pallas_call(
        _conv_body,
        grid=(_NPAD // _BLK,),
        in_specs=[
            pl.BlockSpec((_BLK, _H), lambda i: (i, 0)),
            pl.BlockSpec((_NC, _BLK, _H), lambda i: (0, i, 0)),
            pl.BlockSpec((_H, _H), lambda i: (0, 0)),
            pl.BlockSpec((_H, _H), lambda i: (0, 0)),
            pl.BlockSpec((1, _H), lambda i: (0, 0)),
            pl.BlockSpec((1, _H), lambda i: (0, 0)),
            pl.BlockSpec((1, _H), lambda i: (0, 0)),
        ],
        out_specs=[
            pl.BlockSpec((_BLK, _H), lambda i: (i, 0)),
            pl.BlockSpec((_BLK, _H), lambda i: (i, 0)),
        ],
        out_shape=[
            jax.ShapeDtypeStruct((_NPAD, _H), jnp.float32),
            jax.ShapeDtypeStruct((_NPAD, _H), jnp.bfloat16),
        ],
    )(h, agg2, wr, wn, b.reshape(1, _H), g.reshape(1, _H), bb.reshape(1, _H))


def _pool_final_body(h_ref, bcol_ref, w_ref, b_ref, o_ref, pool_acc, cnt_acc):
    i = pl.program_id(0)

    @pl.when(i == 0)
    def _():
        pool_acc[...] = jnp.zeros_like(pool_acc)
        cnt_acc[...] = jnp.zeros_like(cnt_acc)

    gids = jax.lax.broadcasted_iota(jnp.int32, (_G, 1), 0)
    oh_t = (gids == bcol_ref[0]).astype(jnp.float32)
    hp = jnp.dot(
        oh_t, h_ref[...],
        precision=jax.lax.Precision.HIGHEST,
        preferred_element_type=jnp.float32,
    )
    cp = jnp.dot(
        oh_t, jnp.ones((_BLK, _H), jnp.float32),
        preferred_element_type=jnp.float32,
    )
    pool_acc[...] += hp
    cnt_acc[...] += cp

    @pl.when(i == _NPAD // _BLK - 1)
    def _():
        pooled = pool_acc[...] / jnp.clip(cnt_acc[...], 1.0, None)
        o_ref[...] = (
            jnp.dot(pooled, w_ref[...], preferred_element_type=jnp.float32)
            + b_ref[...]
        )


def _pool_final(h, bcol, w, b):
    return pl.pallas_call(
        _pool_final_body,
        grid=(_NPAD // _BLK,),
        in_specs=[
            pl.BlockSpec((_BLK, _H), lambda i: (i, 0)),
            pl.BlockSpec((1, 1, _BLK), lambda i: (i, 0, 0)),
            pl.BlockSpec((_H, _H), lambda i: (0, 0)),
            pl.BlockSpec((1, _H), lambda i: (0, 0)),
        ],
        out_specs=pl.BlockSpec((_G, _H), lambda i: (0, 0)),
        out_shape=jax.ShapeDtypeStruct((_G, _H), jnp.float32),
        scratch_shapes=[
            pltpu.VMEM((_G, _H), jnp.float32),
            pltpu.VMEM((_G, _H), jnp.float32),
        ],
    )(h, bcol, w, b.reshape(1, _H))


def kernel(x, edge_index, batch, W_in, b_in, W_root, W_neigh, b_conv, ln_g, ln_b, W_out, b_out):
    f32 = jnp.float32
    x_pad = jnp.concatenate([x, jnp.zeros((_NPAD - _N, x.shape[1]), x.dtype)], axis=0)
    pad_ids = _N + (jnp.arange(_EPAD - _E, dtype=jnp.int32) % (_NPAD - _N))
    src_pad = jnp.concatenate([edge_index[0].astype(jnp.int32), pad_ids])
    dst_pad = jnp.concatenate([edge_index[1].astype(jnp.int32), pad_ids])
    eidx = jnp.stack(
        [src_pad.reshape(_NW, _C, _K), dst_pad.reshape(_NW, _C, _K)], axis=2
    )
    bcol = jnp.concatenate(
        [batch.astype(jnp.int32), jnp.full((_NPAD - _N,), _G, jnp.int32)]
    ).reshape(_NPAD // _BLK, 1, _BLK)
    zeros_k = jnp.zeros((_K, _H), f32)

    perm = []
    for cg in range(_H // 32):
        base = 32 * cg
        perm += [base + 2 * k for k in range(16)]
        perm += [base + 2 * k + 1 for k in range(16)]
    perm = jnp.asarray(perm, jnp.int32)

    def pack32(hb):
        return jax.lax.bitcast_convert_type(
            hb.reshape(_NPAD, _H // 2, 2), jnp.int32
        )

    h, hbf = _dense_in(x_pad, W_in, b_in)
    for l in range(_L):
        agg2 = _edge_agg(pack32(hbf), eidx, zeros_k)
        h, hbf = _conv(h, agg2, W_root[l], W_neigh[l][perm, :], b_conv[l],
                       ln_g[l], ln_b[l])
    return _pool_final(h, bcol, W_out, b_out)


# speedup vs baseline: 1.6760x; 1.6760x over previous
import functools

import jax
import jax.numpy as jnp
from jax import lax
from jax.experimental import pallas as pl
from jax.experimental.pallas import tpu as pltpu
from jax.experimental.pallas import tpu_sc as plsc

_N = 10000
_E = 320000
_H = 128
_L = 3
_G = 256

_NPAD = 10240
_NC = 2
_NS = 16
_NW = _NC * _NS
_K = 80
_C = 128
_RB = 4
_GD = 3
_IB = 8
_EPAD = _NW * _C * _K

_PK = 80
_PC = 4
_PROWS = 384
_PSTRIPE = _PROWS // _NS

_STRIPE = _NPAD // _NS


def _sc_mesh():
    return plsc.VectorSubcoreMesh(
        core_axis_name="c", subcore_axis_name="s", num_cores=_NC, num_subcores=_NS
    )


def _edge_agg_body(hb32_hbm, eidx_hbm, zeros_hbm, out_hbm, idx_v, rbf_v, rf32_v,
                   semi, semg, sems, agg_sp):
    c = lax.axis_index("c")
    s = lax.axis_index("s")
    w = c * _NS + s
    row0 = s * _STRIPE
    nz = _STRIPE // _K
    for t in range(nz):
        pltpu.async_copy(zeros_hbm, agg_sp.at[pl.ds(row0 + t * _K, _K)], sems.at[t % 2])

    def idx_load(j, slot):
        pltpu.async_copy(eidx_hbm.at[w, j], idx_v.at[slot], semi.at[slot])

    def idx_wait(j, slot):
        pltpu.make_async_copy(eidx_hbm.at[w, j], idx_v.at[slot], semi.at[slot]).wait()

    def gather(j, u8, r):
        pltpu.async_copy(hb32_hbm.at[idx_v.at[u8, 0]], rbf_v.at[r], semg.at[r])

    def gather_wait(u8, r):
        pltpu.make_async_copy(hb32_hbm.at[idx_v.at[u8, 0]], rbf_v.at[r], semg.at[r]).wait()

    def scatter(u8, p):
        pltpu.async_copy(rf32_v.at[p], agg_sp.at[idx_v.at[u8, 1]], sems.at[p], add=True)

    def scatter_wait(p):
        pltpu.make_async_copy(rf32_v.at[p], agg_sp.at[idx_v.at[0, 1]], sems.at[p]).wait()

    def convert(r, p):
        rb = rbf_v.at[r]
        rf = rf32_v.at[p]
        himask = jnp.int32(-65536)

        @plsc.parallel_loop(0, _K, step=2, unroll=4)
        def crow(row0):
            for k in range(2):
                row = row0 + k
                for g in range(_H // 32):
                    xu = rb[row, pl.ds(16 * g, 16)]
                    rf[row, pl.ds(32 * g, 16)] = jax.lax.bitcast_convert_type(xu << 16, jnp.float32)
                    rf[row, pl.ds(32 * g + 16, 16)] = jax.lax.bitcast_convert_type(xu & himask, jnp.float32)

    def iteration(j, u, skip_b=False, skip_ef=False, do_idx=True):
        idx_wait(j, u)
        if not skip_b:
            scatter_wait(u % 2)
        gather(j, u, u % _RB)
        if do_idx:
            idx_load(j + _RB, (u + _RB) % _IB)
        if not skip_ef:
            um = (u - _GD) % _IB
            gather_wait(um, um % _RB)
            convert(um % _RB, um % 2)
            scatter(um, um % 2)

    for t in range(_RB):
        idx_load(t, t)
    for t in range(_GD):
        iteration(t, t, skip_b=True, skip_ef=True)
    for t in range(nz):
        pltpu.make_async_copy(zeros_hbm, agg_sp.at[pl.ds(row0 + t * _K, _K)], sems.at[t % 2]).wait()
    plsc.subcore_barrier()
    for t in range(_GD, _IB):
        iteration(t, t, skip_b=(t < _RB))

    def body(bb, carry):
        j0 = _IB + _IB * bb
        for u in range(_IB):
            iteration(j0 + u, u)
        return carry

    lax.fori_loop(0, (_C - 2 * _IB) // _IB, body, 0)
    for t in range(_IB):
        j = _C - _IB + t
        iteration(j, t, do_idx=(t < _IB - _RB))
    for t in range(_GD):
        j = _C - _GD + t
        u = j % _IB
        gather_wait(u, u % _RB)
        convert(u % _RB, u % 2)
        scatter(u, u % 2)
    for p in (0, 1, 0, 1):
        scatter_wait(p)
    plsc.subcore_barrier()
    pltpu.sync_copy(agg_sp.at[pl.ds(row0, _STRIPE)], out_hbm.at[c, pl.ds(row0, _STRIPE)])


def _edge_agg(hb32, eidx, zeros_k):
    return pl.kernel(
        _edge_agg_body,
        out_type=jax.ShapeDtypeStruct((_NC, _NPAD, _H), jnp.float32),
        mesh=_sc_mesh(),
        compiler_params=pltpu.CompilerParams(use_tc_tiling_on_sc=False),
        scratch_types=[
            pltpu.VMEM((_IB, 2, _K), jnp.int32),
            pltpu.VMEM((_RB, _K, _H // 2), jnp.int32),
            pltpu.VMEM((2, _K, _H), jnp.float32),
            pltpu.SemaphoreType.DMA((_IB,)),
            pltpu.SemaphoreType.DMA((_RB,)),
            pltpu.SemaphoreType.DMA((2,)),
            pltpu.VMEM_SHARED((_NPAD, _H), jnp.float32),
        ],
    )(hb32, eidx, zeros_k)


_BLK = 1024


def _dense_in_body(x_ref, w_ref, b_ref, o_ref, obf_ref):
    y = (
        jnp.dot(x_ref[...], w_ref[...], preferred_element_type=jnp.float32)
        + b_ref[...]
    )
    o_ref[...] = y
    obf_ref[...] = y.astype(jnp.bfloat16)


def _dense_in(x, w, b):
    return pl.pallas_call(
        _dense_in_body,
        grid=(_NPAD // _BLK,),
        in_specs=[
            pl.BlockSpec((_BLK, _H), lambda i: (i, 0)),
            pl.BlockSpec((_H, _H), lambda i: (0, 0)),
            pl.BlockSpec((1, _H), lambda i: (0, 0)),
        ],
        out_specs=[
            pl.BlockSpec((_BLK, _H), lambda i: (i, 0)),
            pl.BlockSpec((_BLK, _H), lambda i: (i, 0)),
        ],
        out_shape=[
            jax.ShapeDtypeStruct((_NPAD, _H), jnp.float32),
            jax.ShapeDtypeStruct((_NPAD, _H), jnp.bfloat16),
        ],
    )(x, w, b.reshape(1, _H))


def _conv_body(h_ref, a_ref, wr_ref, wn_ref, b_ref, g_ref, bb_ref, o_ref, obf_ref):
    h = h_ref[...]
    a = a_ref[0] + a_ref[1]
    y = (
        jnp.dot(h, wr_ref[...], preferred_element_type=jnp.float32)
        + jnp.dot(a, wn_ref[...], preferred_element_type=jnp.float32)
        + b_ref[...]
    )
    mu = jnp.mean(y, axis=-1, keepdims=True)
    d = y - mu
    var = jnp.mean(d * d, axis=-1, keepdims=True)
    y = d * lax.rsqrt(var + 1e-5) * g_ref[...] + bb_ref[...]
    y = jnp.maximum(y, 0.0)
    o_ref[...] = y
    obf_ref[...] = y.astype(jnp.bfloat16)


def _conv(h, agg2, wr, wn, b, g, bb):
    return pl.pallas_call(
        _conv_body,
        grid=(_NPAD // _BLK,),
        in_specs=[
            pl.BlockSpec((_BLK, _H), lambda i: (i, 0)),
            pl.BlockSpec((_NC, _BLK, _H), lambda i: (0, i, 0)),
            pl.BlockSpec((_H, _H), lambda i: (0, 0)),
            pl.BlockSpec((_H, _H), lambda i: (0, 0)),
            pl.BlockSpec((1, _H), lambda i: (0, 0)),
            pl.BlockSpec((1, _H), lambda i: (0, 0)),
            pl.BlockSpec((1, _H), lambda i: (0, 0)),
        ],
        out_specs=[
            pl.BlockSpec((_BLK, _H), lambda i: (i, 0)),
            pl.BlockSpec((_BLK, _H), lambda i: (i, 0)),
        ],
        out_shape=[
            jax.ShapeDtypeStruct((_NPAD, _H), jnp.float32),
            jax.ShapeDtypeStruct((_NPAD, _H), jnp.bfloat16),
        ],
    )(h, agg2, wr, wn, b.reshape(1, _H), g.reshape(1, _H), bb.reshape(1, _H))


def _pool_final_body(h_ref, bcol_ref, w_ref, b_ref, o_ref, pool_acc, cnt_acc):
    i = pl.program_id(0)

    @pl.when(i == 0)
    def _():
        pool_acc[...] = jnp.zeros_like(pool_acc)
        cnt_acc[...] = jnp.zeros_like(cnt_acc)

    gids = jax.lax.broadcasted_iota(jnp.int32, (_G, 1), 0)
    oh_t = (gids == bcol_ref[0]).astype(jnp.float32)
    hp = jnp.dot(
        oh_t, h_ref[...],
        precision=jax.lax.Precision.HIGHEST,
        preferred_element_type=jnp.float32,
    )
    cp = jnp.dot(
        oh_t, jnp.ones((_BLK, _H), jnp.float32),
        preferred_element_type=jnp.float32,
    )
    pool_acc[...] += hp
    cnt_acc[...] += cp

    @pl.when(i == _NPAD // _BLK - 1)
    def _():
        pooled = pool_acc[...] / jnp.clip(cnt_acc[...], 1.0, None)
        o_ref[...] = (
            jnp.dot(pooled, w_ref[...], preferred_element_type=jnp.float32)
            + b_ref[...]
        )


def _pool_final(h, bcol, w, b):
    return pl.pallas_call(
        _pool_final_body,
        grid=(_NPAD // _BLK,),
        in_specs=[
            pl.BlockSpec((_BLK, _H), lambda i: (i, 0)),
            pl.BlockSpec((1, 1, _BLK), lambda i: (i, 0, 0)),
            pl.BlockSpec((_H, _H), lambda i: (0, 0)),
            pl.BlockSpec((1, _H), lambda i: (0, 0)),
        ],
        out_specs=pl.BlockSpec((_G, _H), lambda i: (0, 0)),
        out_shape=jax.ShapeDtypeStruct((_G, _H), jnp.float32),
        scratch_shapes=[
            pltpu.VMEM((_G, _H), jnp.float32),
            pltpu.VMEM((_G, _H), jnp.float32),
        ],
    )(h, bcol, w, b.reshape(1, _H))


def kernel(x, edge_index, batch, W_in, b_in, W_root, W_neigh, b_conv, ln_g, ln_b, W_out, b_out):
    f32 = jnp.float32
    x_pad = jnp.concatenate([x, jnp.zeros((_NPAD - _N, x.shape[1]), x.dtype)], axis=0)
    pad_ids = _N + (jnp.arange(_EPAD - _E, dtype=jnp.int32) % (_NPAD - _N))
    src_pad = jnp.concatenate([edge_index[0].astype(jnp.int32), pad_ids])
    dst_pad = jnp.concatenate([edge_index[1].astype(jnp.int32), pad_ids])
    eidx = jnp.stack(
        [src_pad.reshape(_NW, _C, _K), dst_pad.reshape(_NW, _C, _K)], axis=2
    )
    bcol = jnp.concatenate(
        [batch.astype(jnp.int32), jnp.full((_NPAD - _N,), _G, jnp.int32)]
    ).reshape(_NPAD // _BLK, 1, _BLK)
    zeros_k = jnp.zeros((_K, _H), f32)

    perm = []
    for cg in range(_H // 32):
        base = 32 * cg
        perm += [base + 2 * k for k in range(16)]
        perm += [base + 2 * k + 1 for k in range(16)]
    perm = jnp.asarray(perm, jnp.int32)

    def pack32(hb):
        return jax.lax.bitcast_convert_type(
            hb.reshape(_NPAD, _H // 2, 2), jnp.int32
        )

    h, hbf = _dense_in(x_pad, W_in, b_in)
    for l in range(_L):
        agg2 = _edge_agg(pack32(hbf), eidx, zeros_k)
        h, hbf = _conv(h, agg2, W_root[l], W_neigh[l][perm, :], b_conv[l],
                       ln_g[l], ln_b[l])
    return _pool_final(h, bcol, W_out, b_out)


# speedup vs baseline: 2.4073x; 1.4363x over previous
import functools

import jax
import jax.numpy as jnp
from jax import lax
from jax.experimental import pallas as pl
from jax.experimental.pallas import tpu as pltpu
from jax.experimental.pallas import tpu_sc as plsc

_N = 10000
_E = 320000
_H = 128
_L = 3
_G = 256

_NPAD = 10240
_NC = 2
_NS = 16
_NW = _NC * _NS
_K = 80
_C = 128
_RB = 4
_GD = 3
_IB = 8
_EPAD = _NW * _C * _K

_PK = 80
_PC = 4
_PROWS = 384
_PSTRIPE = _PROWS // _NS

_STRIPE = _NPAD // _NS


def _sc_mesh():
    return plsc.VectorSubcoreMesh(
        core_axis_name="c", subcore_axis_name="s", num_cores=_NC, num_subcores=_NS
    )


def _edge_agg_body(h_hbm, eidx_hbm, zeros_hbm, out_hbm, idx_v, rows_v,
                   semi, semg, sems, agg_sp):
    c = lax.axis_index("c")
    s = lax.axis_index("s")
    w = c * _NS + s
    row0 = s * _STRIPE
    nz = _STRIPE // _K
    for t in range(nz):
        pltpu.async_copy(zeros_hbm, agg_sp.at[pl.ds(row0 + t * _K, _K)], sems.at[t % _RB])

    def idx_load(j, slot):
        pltpu.async_copy(eidx_hbm.at[w, j], idx_v.at[slot], semi.at[slot])

    def idx_wait(j, slot):
        pltpu.make_async_copy(eidx_hbm.at[w, j], idx_v.at[slot], semi.at[slot]).wait()

    def gather(j, u8, r):
        pltpu.async_copy(h_hbm.at[idx_v.at[u8, 0]], rows_v.at[r], semg.at[r])

    def gather_wait(u8, r):
        pltpu.make_async_copy(h_hbm.at[idx_v.at[u8, 0]], rows_v.at[r], semg.at[r]).wait()

    def scatter(u8, r):
        pltpu.async_copy(rows_v.at[r], agg_sp.at[idx_v.at[u8, 1]], sems.at[r], add=True)

    def scatter_wait(r):
        pltpu.make_async_copy(rows_v.at[r], agg_sp.at[idx_v.at[0, 1]], sems.at[r]).wait()

    def iteration(j, u, skip_b=False, skip_ef=False, do_idx=True):
        idx_wait(j, u)
        if not skip_b:
            scatter_wait(u % _RB)
        gather(j, u, u % _RB)
        if do_idx:
            idx_load(j + _RB, (u + _RB) % _IB)
        if not skip_ef:
            ug = (u - _GD) % _IB
            gather_wait(ug, ug % _RB)
            scatter(ug, ug % _RB)

    for t in range(_RB):
        idx_load(t, t)
    for t in range(_GD):
        iteration(t, t, skip_b=True, skip_ef=True)
    for t in range(nz):
        pltpu.make_async_copy(zeros_hbm, agg_sp.at[pl.ds(row0 + t * _K, _K)], sems.at[t % _RB]).wait()
    plsc.subcore_barrier()
    for t in range(_GD, _IB):
        iteration(t, t, skip_b=(t < _RB))

    def body(bb, carry):
        j0 = _IB + _IB * bb
        for u in range(_IB):
            iteration(j0 + u, u)
        return carry

    lax.fori_loop(0, (_C - 2 * _IB) // _IB, body, 0)
    for t in range(_IB):
        j = _C - _IB + t
        iteration(j, t, do_idx=(t < _IB - _RB))
    for t in range(_GD):
        j = _C - _GD + t
        u = j % _IB
        gather_wait(u, u % _RB)
        scatter(u, u % _RB)
    for r in range(_RB):
        scatter_wait(r)
    plsc.subcore_barrier()
    pltpu.sync_copy(agg_sp.at[pl.ds(row0, _STRIPE)], out_hbm.at[c, pl.ds(row0, _STRIPE)])


def _edge_agg(h, eidx, zeros_k):
    return pl.kernel(
        _edge_agg_body,
        out_type=jax.ShapeDtypeStruct((_NC, _NPAD, _H), jnp.float32),
        mesh=_sc_mesh(),
        compiler_params=pltpu.CompilerParams(use_tc_tiling_on_sc=False),
        scratch_types=[
            pltpu.VMEM((_IB, 2, _K), jnp.int32),
            pltpu.VMEM((_RB, _K, _H), jnp.float32),
            pltpu.SemaphoreType.DMA((_IB,)),
            pltpu.SemaphoreType.DMA((_RB,)),
            pltpu.SemaphoreType.DMA((_RB,)),
            pltpu.VMEM_SHARED((_NPAD, _H), jnp.float32),
        ],
    )(h, eidx, zeros_k)


_BLK = 1024


def _dense_in_body(x_ref, w_ref, b_ref, o_ref):
    o_ref[...] = (
        jnp.dot(x_ref[...], w_ref[...], preferred_element_type=jnp.float32)
        + b_ref[...]
    )


def _dense_in(x, w, b):
    return pl.pallas_call(
        _dense_in_body,
        grid=(_NPAD // _BLK,),
        in_specs=[
            pl.BlockSpec((_BLK, _H), lambda i: (i, 0)),
            pl.BlockSpec((_H, _H), lambda i: (0, 0)),
            pl.BlockSpec((1, _H), lambda i: (0, 0)),
        ],
        out_specs=pl.BlockSpec((_BLK, _H), lambda i: (i, 0)),
        out_shape=jax.ShapeDtypeStruct((_NPAD, _H), jnp.float32),
    )(x, w, b.reshape(1, _H))


def _conv_body(h_ref, a_ref, wr_ref, wn_ref, b_ref, g_ref, bb_ref, o_ref):
    h = h_ref[...]
    a = a_ref[0] + a_ref[1]
    y = (
        jnp.dot(h, wr_ref[...], preferred_element_type=jnp.float32)
        + jnp.dot(a, wn_ref[...], preferred_element_type=jnp.float32)
        + b_ref[...]
    )
    mu = jnp.mean(y, axis=-1, keepdims=True)
    d = y - mu
    var = jnp.mean(d * d, axis=-1, keepdims=True)
    y = d * lax.rsqrt(var + 1e-5) * g_ref[...] + bb_ref[...]
    o_ref[...] = jnp.maximum(y, 0.0)


def _conv(h, agg2, wr, wn, b, g, bb):
    return pl.pallas_call(
        _conv_body,
        grid=(_NPAD // _BLK,),
        in_specs=[
            pl.BlockSpec((_BLK, _H), lambda i: (i, 0)),
            pl.BlockSpec((_NC, _BLK, _H), lambda i: (0, i, 0)),
            pl.BlockSpec((_H, _H), lambda i: (0, 0)),
            pl.BlockSpec((_H, _H), lambda i: (0, 0)),
            pl.BlockSpec((1, _H), lambda i: (0, 0)),
            pl.BlockSpec((1, _H), lambda i: (0, 0)),
            pl.BlockSpec((1, _H), lambda i: (0, 0)),
        ],
        out_specs=pl.BlockSpec((_BLK, _H), lambda i: (i, 0)),
        out_shape=jax.ShapeDtypeStruct((_NPAD, _H), jnp.float32),
    )(h, agg2, wr, wn, b.reshape(1, _H), g.reshape(1, _H), bb.reshape(1, _H))


def _pool_final_body(h_ref, bcol_ref, w_ref, b_ref, o_ref, pool_acc, cnt_acc):
    i = pl.program_id(0)

    @pl.when(i == 0)
    def _():
        pool_acc[...] = jnp.zeros_like(pool_acc)
        cnt_acc[...] = jnp.zeros_like(cnt_acc)

    gids = jax.lax.broadcasted_iota(jnp.int32, (_G, 1), 0)
    oh_t = (gids == bcol_ref[0]).astype(jnp.float32)
    hp = jnp.dot(
        oh_t, h_ref[...],
        precision=jax.lax.Precision.HIGHEST,
        preferred_element_type=jnp.float32,
    )
    cp = jnp.dot(
        oh_t, jnp.ones((_BLK, _H), jnp.float32),
        preferred_element_type=jnp.float32,
    )
    pool_acc[...] += hp
    cnt_acc[...] += cp

    @pl.when(i == _NPAD // _BLK - 1)
    def _():
        pooled = pool_acc[...] / jnp.clip(cnt_acc[...], 1.0, None)
        o_ref[...] = (
            jnp.dot(pooled, w_ref[...], preferred_element_type=jnp.float32)
            + b_ref[...]
        )


def _pool_final(h, bcol, w, b):
    return pl.pallas_call(
        _pool_final_body,
        grid=(_NPAD // _BLK,),
        in_specs=[
            pl.BlockSpec((_BLK, _H), lambda i: (i, 0)),
            pl.BlockSpec((1, 1, _BLK), lambda i: (i, 0, 0)),
            pl.BlockSpec((_H, _H), lambda i: (0, 0)),
            pl.BlockSpec((1, _H), lambda i: (0, 0)),
        ],
        out_specs=pl.BlockSpec((_G, _H), lambda i: (0, 0)),
        out_shape=jax.ShapeDtypeStruct((_G, _H), jnp.float32),
        scratch_shapes=[
            pltpu.VMEM((_G, _H), jnp.float32),
            pltpu.VMEM((_G, _H), jnp.float32),
        ],
    )(h, bcol, w, b.reshape(1, _H))


def kernel(x, edge_index, batch, W_in, b_in, W_root, W_neigh, b_conv, ln_g, ln_b, W_out, b_out):
    f32 = jnp.float32
    x_pad = jnp.concatenate([x, jnp.zeros((_NPAD - _N, x.shape[1]), x.dtype)], axis=0)
    pad_ids = _N + (jnp.arange(_EPAD - _E, dtype=jnp.int32) % (_NPAD - _N))
    src_pad = jnp.concatenate([edge_index[0].astype(jnp.int32), pad_ids])
    dst_pad = jnp.concatenate([edge_index[1].astype(jnp.int32), pad_ids])
    eidx = jnp.stack(
        [src_pad.reshape(_NW, _C, _K), dst_pad.reshape(_NW, _C, _K)], axis=2
    )
    bcol = jnp.concatenate(
        [batch.astype(jnp.int32), jnp.full((_NPAD - _N,), _G, jnp.int32)]
    ).reshape(_NPAD // _BLK, 1, _BLK)
    zeros_k = jnp.zeros((_K, _H), f32)

    h = _dense_in(x_pad, W_in, b_in)
    for l in range(_L):
        agg2 = _edge_agg(h, eidx, zeros_k)
        h = _conv(h, agg2, W_root[l], W_neigh[l], b_conv[l], ln_g[l], ln_b[l])
    return _pool_final(h, bcol, W_out, b_out)


# speedup vs baseline: 2.4754x; 1.0283x over previous
import functools

import jax
import jax.numpy as jnp
from jax import lax
from jax.experimental import pallas as pl
from jax.experimental.pallas import tpu as pltpu
from jax.experimental.pallas import tpu_sc as plsc

_N = 10000
_E = 320000
_H = 128
_L = 3
_G = 256

_NPAD = 10240
_NC = 2
_NS = 16
_NW = _NC * _NS
_K = 80
_C = 128
_RB = 4
_GD = 3
_IB = 8
_EPAD = _NW * _C * _K

_PK = 80
_PC = 4
_PROWS = 384
_PSTRIPE = _PROWS // _NS

_STRIPE = _NPAD // _NS


def _sc_mesh():
    return plsc.VectorSubcoreMesh(
        core_axis_name="c", subcore_axis_name="s", num_cores=_NC, num_subcores=_NS
    )


def _edge_agg_body(h_hbm, eidx_hbm, zeros_hbm, out_hbm, idx_v, rows_v,
                   semi, semg, sems, agg_sp):
    c = lax.axis_index("c")
    s = lax.axis_index("s")
    w = c * _NS + s
    row0 = s * _STRIPE
    nz = _STRIPE // _K
    for t in range(nz):
        pltpu.async_copy(zeros_hbm, agg_sp.at[pl.ds(row0 + t * _K, _K)], sems.at[t % _RB])

    def idx_load(j, slot):
        pltpu.async_copy(eidx_hbm.at[w, j], idx_v.at[slot], semi.at[slot])

    def idx_wait(j, slot):
        pltpu.make_async_copy(eidx_hbm.at[w, j], idx_v.at[slot], semi.at[slot]).wait()

    def gather(j, u8, r):
        pltpu.async_copy(h_hbm.at[idx_v.at[u8, 0]], rows_v.at[r], semg.at[r])

    def gather_wait(u8, r):
        pltpu.make_async_copy(h_hbm.at[idx_v.at[u8, 0]], rows_v.at[r], semg.at[r]).wait()

    def scatter(u8, r):
        pltpu.async_copy(rows_v.at[r], agg_sp.at[idx_v.at[u8, 1]], sems.at[r], add=True)

    def scatter_wait(r):
        pltpu.make_async_copy(rows_v.at[r], agg_sp.at[idx_v.at[0, 1]], sems.at[r]).wait()

    def iteration(j, u, skip_b=False, skip_ef=False, do_idx=True):
        idx_wait(j, u)
        if not skip_b:
            scatter_wait(u % _RB)
        gather(j, u, u % _RB)
        if do_idx:
            idx_load(j + _RB, (u + _RB) % _IB)
        if not skip_ef:
            ug = (u - _GD) % _IB
            gather_wait(ug, ug % _RB)
            scatter(ug, ug % _RB)

    for t in range(_RB):
        idx_load(t, t)
    for t in range(_GD):
        iteration(t, t, skip_b=True, skip_ef=True)
    for t in range(nz):
        pltpu.make_async_copy(zeros_hbm, agg_sp.at[pl.ds(row0 + t * _K, _K)], sems.at[t % _RB]).wait()
    plsc.subcore_barrier()
    for t in range(_GD, _IB):
        iteration(t, t, skip_b=(t < _RB))

    def body(bb, carry):
        j0 = _IB + _IB * bb
        for u in range(_IB):
            iteration(j0 + u, u)
        return carry

    lax.fori_loop(0, (_C - 2 * _IB) // _IB, body, 0)
    for t in range(_IB):
        j = _C - _IB + t
        iteration(j, t, do_idx=(t < _IB - _RB))
    for t in range(_GD):
        j = _C - _GD + t
        u = j % _IB
        gather_wait(u, u % _RB)
        scatter(u, u % _RB)
    for r in range(_RB):
        scatter_wait(r)
    plsc.subcore_barrier()
    pltpu.sync_copy(agg_sp.at[pl.ds(row0, _STRIPE)], out_hbm.at[c, pl.ds(row0, _STRIPE)])


def _edge_agg(h, eidx, zeros_k):
    return pl.kernel(
        _edge_agg_body,
        out_type=jax.ShapeDtypeStruct((_NC, _NPAD, _H), jnp.float32),
        mesh=_sc_mesh(),
        scratch_types=[
            pltpu.VMEM((_IB, 2, _K), jnp.int32),
            pltpu.VMEM((_RB, _K, _H), jnp.float32),
            pltpu.SemaphoreType.DMA((_IB,)),
            pltpu.SemaphoreType.DMA((_RB,)),
            pltpu.SemaphoreType.DMA((_RB,)),
            pltpu.VMEM_SHARED((_NPAD, _H), jnp.float32),
        ],
    )(h, eidx, zeros_k)


_BLK = 1024


def _dense_in_body(x_ref, w_ref, b_ref, o_ref):
    o_ref[...] = (
        jnp.dot(x_ref[...], w_ref[...], preferred_element_type=jnp.float32)
        + b_ref[...]
    )


def _dense_in(x, w, b):
    return pl.pallas_call(
        _dense_in_body,
        grid=(_NPAD // _BLK,),
        in_specs=[
            pl.BlockSpec((_BLK, _H), lambda i: (i, 0)),
            pl.BlockSpec((_H, _H), lambda i: (0, 0)),
            pl.BlockSpec((1, _H), lambda i: (0, 0)),
        ],
        out_specs=pl.BlockSpec((_BLK, _H), lambda i: (i, 0)),
        out_shape=jax.ShapeDtypeStruct((_NPAD, _H), jnp.float32),
    )(x, w, b.reshape(1, _H))


def _conv_body(h_ref, a_ref, wr_ref, wn_ref, b_ref, g_ref, bb_ref, o_ref):
    h = h_ref[...]
    a = a_ref[0] + a_ref[1]
    y = (
        jnp.dot(h, wr_ref[...], preferred_element_type=jnp.float32)
        + jnp.dot(a, wn_ref[...], preferred_element_type=jnp.float32)
        + b_ref[...]
    )
    mu = jnp.mean(y, axis=-1, keepdims=True)
    d = y - mu
    var = jnp.mean(d * d, axis=-1, keepdims=True)
    y = d * lax.rsqrt(var + 1e-5) * g_ref[...] + bb_ref[...]
    o_ref[...] = jnp.maximum(y, 0.0)


def _conv(h, agg2, wr, wn, b, g, bb):
    return pl.pallas_call(
        _conv_body,
        grid=(_NPAD // _BLK,),
        in_specs=[
            pl.BlockSpec((_BLK, _H), lambda i: (i, 0)),
            pl.BlockSpec((_NC, _BLK, _H), lambda i: (0, i, 0)),
            pl.BlockSpec((_H, _H), lambda i: (0, 0)),
            pl.BlockSpec((_H, _H), lambda i: (0, 0)),
            pl.BlockSpec((1, _H), lambda i: (0, 0)),
            pl.BlockSpec((1, _H), lambda i: (0, 0)),
            pl.BlockSpec((1, _H), lambda i: (0, 0)),
        ],
        out_specs=pl.BlockSpec((_BLK, _H), lambda i: (i, 0)),
        out_shape=jax.ShapeDtypeStruct((_NPAD, _H), jnp.float32),
    )(h, agg2, wr, wn, b.reshape(1, _H), g.reshape(1, _H), bb.reshape(1, _H))


def _pool_final_body(h_ref, bcol_ref, w_ref, b_ref, o_ref, pool_acc, cnt_acc):
    i = pl.program_id(0)

    @pl.when(i == 0)
    def _():
        pool_acc[...] = jnp.zeros_like(pool_acc)
        cnt_acc[...] = jnp.zeros_like(cnt_acc)

    gids = jax.lax.broadcasted_iota(jnp.int32, (_G, 1), 0)
    oh_t = (gids == bcol_ref[0]).astype(jnp.float32)
    hp = jnp.dot(
        oh_t, h_ref[...],
        precision=jax.lax.Precision.HIGHEST,
        preferred_element_type=jnp.float32,
    )
    cp = jnp.dot(
        oh_t, jnp.ones((_BLK, _H), jnp.float32),
        preferred_element_type=jnp.float32,
    )
    pool_acc[...] += hp
    cnt_acc[...] += cp

    @pl.when(i == _NPAD // _BLK - 1)
    def _():
        pooled = pool_acc[...] / jnp.clip(cnt_acc[...], 1.0, None)
        o_ref[...] = (
            jnp.dot(pooled, w_ref[...], preferred_element_type=jnp.float32)
            + b_ref[...]
        )


def _pool_final(h, bcol, w, b):
    return pl.pallas_call(
        _pool_final_body,
        grid=(_NPAD // _BLK,),
        in_specs=[
            pl.BlockSpec((_BLK, _H), lambda i: (i, 0)),
            pl.BlockSpec((1, 1, _BLK), lambda i: (i, 0, 0)),
            pl.BlockSpec((_H, _H), lambda i: (0, 0)),
            pl.BlockSpec((1, _H), lambda i: (0, 0)),
        ],
        out_specs=pl.BlockSpec((_G, _H), lambda i: (0, 0)),
        out_shape=jax.ShapeDtypeStruct((_G, _H), jnp.float32),
        scratch_shapes=[
            pltpu.VMEM((_G, _H), jnp.float32),
            pltpu.VMEM((_G, _H), jnp.float32),
        ],
    )(h, bcol, w, b.reshape(1, _H))


def kernel(x, edge_index, batch, W_in, b_in, W_root, W_neigh, b_conv, ln_g, ln_b, W_out, b_out):
    f32 = jnp.float32
    x_pad = jnp.concatenate([x, jnp.zeros((_NPAD - _N, x.shape[1]), x.dtype)], axis=0)
    pad_ids = _N + (jnp.arange(_EPAD - _E, dtype=jnp.int32) % (_NPAD - _N))
    src_pad = jnp.concatenate([edge_index[0].astype(jnp.int32), pad_ids])
    dst_pad = jnp.concatenate([edge_index[1].astype(jnp.int32), pad_ids])
    eidx = jnp.stack(
        [src_pad.reshape(_NW, _C, _K), dst_pad.reshape(_NW, _C, _K)], axis=2
    )
    bcol = jnp.concatenate(
        [batch.astype(jnp.int32), jnp.full((_NPAD - _N,), _G, jnp.int32)]
    ).reshape(_NPAD // _BLK, 1, _BLK)
    zeros_k = jnp.zeros((_K, _H), f32)

    h = _dense_in(x_pad, W_in, b_in)
    for l in range(_L):
        agg2 = _edge_agg(h, eidx, zeros_k)
        h = _conv(h, agg2, W_root[l], W_neigh[l], b_conv[l], ln_g[l], ln_b[l])
    return _pool_final(h, bcol, W_out, b_out)
